# baseline (speedup 1.0000x reference)
import functools

import jax
import jax.numpy as jnp
from jax import lax
from jax.experimental import pallas as pl
from jax.experimental.pallas import tpu as pltpu
from jax.experimental.pallas import tpu_sc as plsc

BATCH = 4096
SEQ = 200
DIM = 64
NLANE = 16
NC = 2
NS = 16
NW = NC * NS
ROWS = BATCH * SEQ
RPW = ROWS // NW
CH = SEQ
NCHUNK = RPW // CH
NBUF = 4
SPLIT = 104


def _sc_embed(x_flat, class_table, pos_table):
    mesh = plsc.VectorSubcoreMesh(core_axis_name="c", subcore_axis_name="s")

    @functools.partial(
        pl.kernel,
        out_type=jax.ShapeDtypeStruct((ROWS, DIM), jnp.float32),
        mesh=mesh,
        compiler_params=pltpu.CompilerParams(use_tc_tiling_on_sc=False),
        scratch_types=[
            pltpu.VMEM((RPW,), jnp.int32),
            pltpu.VMEM((SEQ, DIM), jnp.float32),
            [pltpu.VMEM((CH, DIM), jnp.float32) for _ in range(NBUF)],
            [pltpu.SemaphoreType.DMA for _ in range(NBUF)],
        ],
    )
    def kern(x_hbm, tab_hbm, pos_hbm, out_hbm, idx_v, pos_v, bufs, sems):
        wid = lax.axis_index("s") * NC + lax.axis_index("c")
        base = pl.multiple_of(wid * RPW, RPW)
        pltpu.sync_copy(x_hbm.at[pl.ds(base, RPW)], idx_v)
        pltpu.sync_copy(pos_hbm, pos_v)

        def prefill_pos(b):
            @plsc.parallel_loop(0, CH, unroll=4)
            def _(r):
                for j in range(DIM // NLANE):
                    sl = pl.ds(j * NLANE, NLANE)
                    bufs[b][r, sl] = pos_v[r, sl]

        def start_gather(c, b):
            off = pl.multiple_of(c * CH, 8)
            pltpu.async_copy(
                tab_hbm.at[idx_v.at[pl.ds(off, SPLIT)]],
                bufs[b].at[pl.ds(0, SPLIT)], sems[b], add=True)
            off2 = pl.multiple_of(off + SPLIT, 8)
            pltpu.async_copy(
                tab_hbm.at[idx_v.at[pl.ds(off2, CH - SPLIT)]],
                bufs[b].at[pl.ds(SPLIT, CH - SPLIT)], sems[b], add=True)

        def wait_gather(b):
            pltpu.make_async_copy(
                tab_hbm.at[pl.ds(0, SPLIT)],
                bufs[b].at[pl.ds(0, SPLIT)], sems[b]).wait()
            pltpu.make_async_copy(
                tab_hbm.at[pl.ds(0, CH - SPLIT)],
                bufs[b].at[pl.ds(SPLIT, CH - SPLIT)], sems[b]).wait()

        for b in range(NBUF - 1):
            prefill_pos(b)
            start_gather(b, b)

        def chunk_step(c, b):
            wait_gather(b)
            nxt = c + (NBUF - 1)

            @pl.when(nxt < NCHUNK)
            def _():
                bn = (b + NBUF - 1) % NBUF
                prefill_pos(bn)
                start_gather(nxt, bn)

            pltpu.sync_copy(bufs[b], out_hbm.at[pl.ds(base + c * CH, CH)])

        @pl.loop(0, NCHUNK // NBUF)
        def _(g):
            for b in range(NBUF):
                chunk_step(g * NBUF + b, b)

    return kern(x_flat, class_table, pos_table)


def kernel(x, class_table, pos_table):
    x_flat = x.reshape(-1).astype(jnp.int32)
    out = _sc_embed(x_flat, class_table, pos_table)
    return out.reshape(BATCH, SEQ, DIM)


# speedup vs baseline: 1.0018x; 1.0018x over previous
import functools

import jax
import jax.numpy as jnp
from jax import lax
from jax.experimental import pallas as pl
from jax.experimental.pallas import tpu as pltpu
from jax.experimental.pallas import tpu_sc as plsc

BATCH = 4096
SEQ = 200
DIM = 64
NLANE = 16
NC = 2
NS = 16
NW = NC * NS
ROWS = BATCH * SEQ
RPW = ROWS // NW
CH = SEQ
NCHUNK = RPW // CH
NBUF = 4
SPLIT = 104


def _sc_embed(x_flat, class_table, pos_table):
    mesh = plsc.VectorSubcoreMesh(core_axis_name="c", subcore_axis_name="s")

    @functools.partial(
        pl.kernel,
        out_type=jax.ShapeDtypeStruct((BATCH, SEQ, DIM), jnp.float32),
        mesh=mesh,
        compiler_params=pltpu.CompilerParams(use_tc_tiling_on_sc=False),
        scratch_types=[
            pltpu.VMEM((RPW,), jnp.int32),
            pltpu.VMEM((SEQ, DIM), jnp.float32),
            [pltpu.VMEM((CH, DIM), jnp.float32) for _ in range(NBUF)],
            [pltpu.SemaphoreType.DMA for _ in range(NBUF)],
        ],
    )
    def kern(x_hbm, tab_hbm, pos_hbm, out_hbm, idx_v, pos_v, bufs, sems):
        wid = lax.axis_index("s") * NC + lax.axis_index("c")
        base = pl.multiple_of(wid * RPW, RPW)
        pltpu.sync_copy(x_hbm.at[pl.ds(base, RPW)], idx_v)
        pltpu.sync_copy(pos_hbm, pos_v)

        def prefill_pos(b):
            @plsc.parallel_loop(0, CH, unroll=4)
            def _(r):
                for j in range(DIM // NLANE):
                    sl = pl.ds(j * NLANE, NLANE)
                    bufs[b][r, sl] = pos_v[r, sl]

        def start_gather(c, b):
            off = pl.multiple_of(c * CH, 8)
            pltpu.async_copy(
                tab_hbm.at[idx_v.at[pl.ds(off, SPLIT)]],
                bufs[b].at[pl.ds(0, SPLIT)], sems[b], add=True)
            off2 = pl.multiple_of(off + SPLIT, 8)
            pltpu.async_copy(
                tab_hbm.at[idx_v.at[pl.ds(off2, CH - SPLIT)]],
                bufs[b].at[pl.ds(SPLIT, CH - SPLIT)], sems[b], add=True)

        def wait_gather(b):
            pltpu.make_async_copy(
                tab_hbm.at[pl.ds(0, SPLIT)],
                bufs[b].at[pl.ds(0, SPLIT)], sems[b]).wait()
            pltpu.make_async_copy(
                tab_hbm.at[pl.ds(0, CH - SPLIT)],
                bufs[b].at[pl.ds(SPLIT, CH - SPLIT)], sems[b]).wait()

        for b in range(NBUF - 1):
            prefill_pos(b)
            start_gather(b, b)

        def chunk_step(c, b):
            wait_gather(b)
            nxt = c + (NBUF - 1)

            @pl.when(nxt < NCHUNK)
            def _():
                bn = (b + NBUF - 1) % NBUF
                prefill_pos(bn)
                start_gather(nxt, bn)

            pltpu.sync_copy(bufs[b], out_hbm.at[wid * NCHUNK + c])

        @pl.loop(0, NCHUNK // NBUF)
        def _(g):
            for b in range(NBUF):
                chunk_step(g * NBUF + b, b)

    return kern(x_flat, class_table, pos_table)


def kernel(x, class_table, pos_table):
    x_flat = x.reshape(-1).astype(jnp.int32)
    return _sc_embed(x_flat, class_table, pos_table)
